# baseline (speedup 1.0000x reference)
import jax
import jax.numpy as jnp
from jax.experimental import pallas as pl


def _body(x1_ref, x2_ref, o1_ref, o2_ref):
    o1_ref[:, 0, :] = x2_ref[:, 0, :]
    o1_ref[:, 1, :] = x1_ref[:, 1, :]
    o2_ref[:, 0, :] = x1_ref[:, 0, :]
    o2_ref[:, 1, :] = x2_ref[:, 1, :]


def kernel(x1, x2):
    N, C, H, W = x1.shape
    R = N * C // 2
    Wf = H * W
    a = x1.reshape(R, 2, Wf)
    b = x2.reshape(R, 2, Wf)
    BP = 4
    spec = pl.BlockSpec((BP, 2, Wf), lambda i: (i, 0, 0))
    o1, o2 = pl.pallas_call(
        _body,
        grid=(R // BP,),
        in_specs=[spec, spec],
        out_specs=[spec, spec],
        out_shape=[jax.ShapeDtypeStruct((R, 2, Wf), jnp.float32)] * 2,
    )(a, b)
    return o1.reshape(N, C, H, W), o2.reshape(N, C, H, W)


# speedup vs baseline: 5.3007x; 5.3007x over previous
import jax
import jax.numpy as jnp
from jax.experimental import pallas as pl


def _body(x1_ref, x2_ref, o1_ref, o2_ref):
    o1_ref[:, 0] = x2_ref[:, 0]
    o1_ref[:, 1] = x1_ref[:, 1]
    o2_ref[:, 0] = x1_ref[:, 0]
    o2_ref[:, 1] = x2_ref[:, 1]


def kernel(x1, x2):
    N, C, H, W = x1.shape
    R = N * C // 2
    a = x1.reshape(R, 2, H, W)
    b = x2.reshape(R, 2, H, W)
    BP = 4
    spec = pl.BlockSpec((BP, 2, H, W), lambda i: (i, 0, 0, 0))
    o1, o2 = pl.pallas_call(
        _body,
        grid=(R // BP,),
        in_specs=[spec, spec],
        out_specs=[spec, spec],
        out_shape=[jax.ShapeDtypeStruct((R, 2, H, W), jnp.float32)] * 2,
    )(a, b)
    return o1.reshape(N, C, H, W), o2.reshape(N, C, H, W)
